# baseline (speedup 1.0000x reference)
import functools

import jax
import jax.numpy as jnp
from jax import lax
from jax.experimental import pallas as pl
from jax.experimental.pallas import tpu as pltpu
from jax.experimental.pallas import tpu_sc as plsc

NC = 2
NS = 16
NW = NC * NS

B_TOTAL = 16384 * 200
D = 64
B_PER_W = B_TOTAL // NW

CHUNK = 512
K = CHUNK // 128
N_STEPS = B_PER_W // CHUNK


def _emb_body(idx_hbm, table_hbm, out_hbm, idx_v, rows_v, gat_sem):
  wid = lax.axis_index("s") * NC + lax.axis_index("c")
  base_row = wid * (B_PER_W // 128)

  def step(s, carry):
    row0 = base_row + s * K
    pltpu.sync_copy(idx_hbm.at[pl.ds(row0, K)], idx_v)
    copies = [
        pltpu.async_copy(
            table_hbm.at[idx_v.at[j]],
            rows_v.at[pl.ds(j * 128, 128)],
            gat_sem,
        )
        for j in range(K)
    ]
    for cp in copies:
      cp.wait()
    pltpu.sync_copy(rows_v, out_hbm.at[pl.ds(row0 * 128, CHUNK)])
    return carry

  lax.fori_loop(0, N_STEPS, step, 0)


@jax.jit
def _gather(idx2d, weight):
  mesh = plsc.VectorSubcoreMesh(
      core_axis_name="c", subcore_axis_name="s", num_cores=NC, num_subcores=NS
  )
  f = pl.kernel(
      _emb_body,
      out_type=jax.ShapeDtypeStruct((B_TOTAL, D), jnp.float32),
      mesh=mesh,
      scratch_types=[
          pltpu.VMEM((K, 128), jnp.int32),
          pltpu.VMEM((CHUNK, D), jnp.float32),
          pltpu.SemaphoreType.DMA,
      ],
      compiler_params=pltpu.CompilerParams(use_tc_tiling_on_sc=False),
  )
  return f(idx2d, weight)


def kernel(input, weight):
  idx2d = input.reshape(B_TOTAL // 128, 128)
  out = _gather(idx2d, weight)
  return out.reshape(input.shape + (D,))
